# baseline (speedup 1.0000x reference)
import jax
import jax.numpy as jnp
from jax import lax
from jax.experimental import pallas as pl
from jax.experimental.pallas import tpu as pltpu
from jax.experimental.pallas import tpu_sc as plsc

VOCAB = 100000
EMBED = 128
BATCH = 1024
HIST = 200

_INFO = plsc.get_sparse_core_info()
_NC = _INFO.num_cores
_NS = _INFO.num_subcores
_NW = _NC * _NS

_B = BATCH * HIST
_B_PER_W = _B // _NW
_CHUNK = 128
_N_CHUNKS = _B_PER_W // _CHUNK
_NB = 5
_NSH = 2
_STEP = 10


def _emb_kernel(table_hbm, idx_hbm, out_hbm, idx_all, shared, *rest):
    rows = rest[:_NB]
    gsem = rest[_NB:2 * _NB]
    wsem = rest[2 * _NB:3 * _NB]
    hsem = rest[3 * _NB:3 * _NB + _NSH]

    sid = lax.axis_index("s")
    wid = sid * _NC + lax.axis_index("c")
    base = wid * _B_PER_W

    pltpu.sync_copy(idx_hbm.at[pl.ds(base, _B_PER_W)], idx_all)

    def gather(slot, g):
        pltpu.async_copy(
            table_hbm.at[idx_all.at[pl.ds(g * _CHUNK, _CHUNK)]],
            rows[slot], gsem[slot])

    def hbm_write(g, s):
        pltpu.async_copy(
            shared.at[sid, s], out_hbm.at[pl.ds(base + g * _CHUNK, _CHUNK)],
            hsem[s])

    gather(0, 0)
    gather(1, 1)

    def body(go, _):
        for j in range(_STEP):
            b = j % _NB
            s = j % _NSH
            b1 = (j - 1) % _NB
            s1 = (j - 1) % _NSH
            g = go + j

            @pl.when(g >= 1)
            def _():
                pltpu.make_async_copy(
                    rows[b1], shared.at[sid, s1], wsem[b1]).wait()
                hbm_write(g - 1, s1)

            @pl.when(g + 2 < _N_CHUNKS)
            def _():
                gather((j + 2) % _NB, g + 2)

            pltpu.make_async_copy(
                table_hbm.at[idx_all.at[pl.ds(0, _CHUNK)]],
                rows[b], gsem[b]).wait()

            @pl.when(g >= _NSH)
            def _():
                pltpu.make_async_copy(
                    shared.at[sid, s],
                    out_hbm.at[pl.ds(0, _CHUNK)], hsem[s]).wait()
            pltpu.async_copy(rows[b], shared.at[sid, s], wsem[b])
        return ()

    lax.fori_loop(0, _N_CHUNKS // _STEP, lambda i, c: body(i * _STEP, c), (),
                  unroll=False)

    gl = _N_CHUNKS - 1
    bl = gl % _NB
    sl = gl % _NSH
    pltpu.make_async_copy(rows[bl], shared.at[sid, sl], wsem[bl]).wait()
    hbm_write(gl, sl)
    for s in range(_NSH):
        pltpu.make_async_copy(
            shared.at[sid, s], out_hbm.at[pl.ds(0, _CHUNK)], hsem[s]).wait()


@jax.jit
def _run(x_flat, weight):
    mesh = plsc.VectorSubcoreMesh(core_axis_name="c", subcore_axis_name="s")
    scratch = [pltpu.VMEM((_B_PER_W,), jnp.int32),
               pltpu.VMEM_SHARED((_NS, _NSH, _CHUNK, EMBED), jnp.float32)]
    scratch += [pltpu.VMEM((_CHUNK, EMBED), jnp.float32) for _ in range(_NB)]
    scratch += [pltpu.SemaphoreType.DMA for _ in range(2 * _NB + _NSH)]
    return pl.kernel(
        _emb_kernel,
        out_type=jax.ShapeDtypeStruct((_B, EMBED), jnp.float32),
        mesh=mesh,
        scratch_types=scratch,
    )(weight, x_flat)


def kernel(x, weight):
    out = _run(x.reshape(_B).astype(jnp.int32), weight)
    return out.reshape(BATCH, HIST, EMBED)
